# baseline (speedup 1.0000x reference)
import jax
import jax.numpy as jnp
from jax import lax
from jax.experimental import pallas as pl
from jax.experimental.pallas import tpu as pltpu
from jax.experimental.pallas import tpu_sc as plsc

B = 16384
D = 64
NC, NS, L = 2, 16, 16
NW = NC * NS
BPW = B // NW
CHUNK = 128
NCHUNK = BPW // CHUNK
TP = L + 1


def _sc_body(cust_hbm, arti_hbm, artj_hbm, wc_hbm, wa_hbm,
             out_i_hbm, out_j_hbm,
             idx_c, idx_i, idx_j, c_rows, ai_rows, aj_rows,
             dots_i, dots_j, ti, tj, sem0, sem1, sem2, sem3):
    wid = lax.axis_index("s") * NC + lax.axis_index("c")
    row4 = wid * NCHUNK

    pltpu.sync_copy(cust_hbm.at[pl.ds(row4, NCHUNK)], idx_c)
    pltpu.sync_copy(arti_hbm.at[pl.ds(row4, NCHUNK)], idx_i)
    pltpu.sync_copy(artj_hbm.at[pl.ds(row4, NCHUNK)], idx_j)

    sems = (sem0, sem1, sem2, sem3)
    copies = []
    for k in range(NCHUNK):
        dst = pl.ds(k * CHUNK, CHUNK)
        copies.append((
            pltpu.async_copy(wc_hbm.at[idx_c.at[k]], c_rows.at[dst], sems[k]),
            pltpu.async_copy(wa_hbm.at[idx_i.at[k]], ai_rows.at[dst], sems[k]),
            pltpu.async_copy(wa_hbm.at[idx_j.at[k]], aj_rows.at[dst], sems[k]),
        ))

    scat_base = lax.iota(jnp.int32, L) * TP

    def block_body(blk, _):
        r0 = blk * L
        for r_local in range(L):
            r = r0 + r_local
            sl0 = pl.ds(0 * L, L)
            cv = c_rows[r, sl0]
            s_i = cv * ai_rows[r, sl0]
            s_j = cv * aj_rows[r, sl0]
            for seg in range(1, D // L):
                sl = pl.ds(seg * L, L)
                cv = c_rows[r, sl]
                s_i = s_i + cv * ai_rows[r, sl]
                s_j = s_j + cv * aj_rows[r, sl]
            idx = scat_base + r_local
            plsc.store_scatter(ti, [idx], s_i)
            plsc.store_scatter(tj, [idx], s_j)
        acc_i = ti[pl.ds(0, L)]
        acc_j = tj[pl.ds(0, L)]
        for l in range(1, L):
            acc_i = acc_i + ti[pl.ds(l * TP, L)]
            acc_j = acc_j + tj[pl.ds(l * TP, L)]
        out_sl = pl.ds(r0, L)
        dots_i[out_sl] = 1.0 / (1.0 + jnp.exp(-acc_i))
        dots_j[out_sl] = 1.0 / (1.0 + jnp.exp(-acc_j))
        return 0

    blocks_per_chunk = CHUNK // L
    for k in range(NCHUNK):
        for c in copies[k]:
            c.wait()
        lax.fori_loop(k * blocks_per_chunk, (k + 1) * blocks_per_chunk,
                      block_body, 0)

    out = pl.ds(wid * BPW, BPW)
    pltpu.sync_copy(dots_i, out_i_hbm.at[out])
    pltpu.sync_copy(dots_j, out_j_hbm.at[out])


@jax.jit
def _sc_call(cust2d, arti2d, artj2d, wc, wa):
    mesh = plsc.VectorSubcoreMesh(core_axis_name="c", subcore_axis_name="s")
    f = pl.kernel(
        _sc_body,
        out_type=(
            jax.ShapeDtypeStruct((B,), jnp.float32),
            jax.ShapeDtypeStruct((B,), jnp.float32),
        ),
        mesh=mesh,
        compiler_params=pltpu.CompilerParams(
            needs_layout_passes=False, use_tc_tiling_on_sc=False),
        scratch_types=[
            pltpu.VMEM((NCHUNK, CHUNK), jnp.int32),
            pltpu.VMEM((NCHUNK, CHUNK), jnp.int32),
            pltpu.VMEM((NCHUNK, CHUNK), jnp.int32),
            pltpu.VMEM((BPW, D), jnp.float32),
            pltpu.VMEM((BPW, D), jnp.float32),
            pltpu.VMEM((BPW, D), jnp.float32),
            pltpu.VMEM((BPW,), jnp.float32),
            pltpu.VMEM((BPW,), jnp.float32),
            pltpu.VMEM((L * TP,), jnp.float32),
            pltpu.VMEM((L * TP,), jnp.float32),
            pltpu.SemaphoreType.DMA,
            pltpu.SemaphoreType.DMA,
            pltpu.SemaphoreType.DMA,
            pltpu.SemaphoreType.DMA,
        ],
    )
    return f(cust2d, arti2d, artj2d, wc, wa)


def kernel(customer, article_i, article_j, W_customer, W_article):
    cust2d = customer.reshape(B // CHUNK, CHUNK)
    arti2d = article_i.reshape(B // CHUNK, CHUNK)
    artj2d = article_j.reshape(B // CHUNK, CHUNK)
    return _sc_call(cust2d, arti2d, artj2d, W_customer, W_article)
